# Initial kernel scaffold; baseline (speedup 1.0000x reference)
import jax
import jax.numpy as jnp
from jax.experimental import pallas as pl


def kernel(graph1, graph2, W1, b1, gamma, beta, W2, b2):
    raise NotImplementedError("write your pallas kernel here")


# speedup vs baseline: 3.0627x; 3.0627x over previous
import functools
import math

import jax
import jax.numpy as jnp
from jax import lax
from jax.experimental import pallas as pl

N = 100000
D = 128
TOP_K = 1024
ROWS = 800
PAD = ROWS * D - N
DIST_BLK = 4000
DIST_GRID = N // DIST_BLK


def _dist_body(g1_ref, g2_ref, out_ref):
    x = g1_ref[...] - g2_ref[...] + 1e-6
    ones = jnp.ones((1, D), jnp.float32)
    s = lax.dot_general(ones, x * x, (((1,), (1,)), ((), ())),
                        preferred_element_type=jnp.float32)
    out_ref[...] = jnp.sqrt(s)[None]


def _distances(g1, g2):
    return pl.pallas_call(
        _dist_body,
        grid=(DIST_GRID,),
        in_specs=[
            pl.BlockSpec((DIST_BLK, D), lambda i: (i, 0)),
            pl.BlockSpec((DIST_BLK, D), lambda i: (i, 0)),
        ],
        out_specs=pl.BlockSpec((1, 1, DIST_BLK), lambda i: (i, 0, 0)),
        out_shape=jax.ShapeDtypeStruct((DIST_GRID, 1, DIST_BLK), jnp.float32),
    )(g1, g2)


def _flat_roll(x, s, sent):
    r, c = x.shape
    if s >= c:
        m = s // c
        assert s % c == 0
        return jnp.concatenate(
            [x[m:, :], jnp.full((m, c), sent, x.dtype)], axis=0)
    nxt = jnp.concatenate([x[1:, :], jnp.full((1, c), sent, x.dtype)], axis=0)
    return jnp.concatenate([x[:, s:], nxt[:, :s]], axis=1)


def _topk_mlp_body(sim_ref, w1_ref, b1_ref, gam_ref, bet_ref, w2_ref, b2_ref,
                   out_ref):
    sim = sim_ref[...]
    bits = lax.bitcast_convert_type(sim, jnp.int32)

    def bs(_, carry):
        lo, hi = carry
        mid = lo + (hi - lo + jnp.int32(1)) // 2
        cnt = jnp.sum((bits >= mid).astype(jnp.int32))
        ge = cnt >= TOP_K
        return (jnp.where(ge, mid, lo), jnp.where(ge, hi, mid - 1))

    lo, _ = lax.fori_loop(0, 31, bs, (jnp.int32(0), jnp.int32(0x7F800000)))
    t_f = lax.bitcast_convert_type(lo, jnp.float32)

    mask = bits > lo
    maskf = mask.astype(jnp.float32)
    m = jnp.sum(mask.astype(jnp.int32))

    li = lax.broadcasted_iota(jnp.int32, (D, D), 0)
    lj = lax.broadcasted_iota(jnp.int32, (D, D), 1)
    upper = (li <= lj).astype(jnp.float32)
    incl = lax.dot_general(maskf, upper, (((1,), (0,)), ((), ())),
                           preferred_element_type=jnp.float32)
    rowtot = incl[:, D - 1:D]
    ri = lax.broadcasted_iota(jnp.int32, (ROWS, ROWS), 0)
    rj = lax.broadcasted_iota(jnp.int32, (ROWS, ROWS), 1)
    ltri = (rj < ri).astype(jnp.float32)
    offs = lax.dot_general(ltri, rowtot, (((1,), (0,)), ((), ())),
                           preferred_element_type=jnp.float32)
    p = (offs + incl - maskf).astype(jnp.int32)

    rows_i = lax.broadcasted_iota(jnp.int32, (ROWS, D), 0)
    lanes_i = lax.broadcasted_iota(jnp.int32, (ROWS, D), 1)
    flat = rows_i * D + lanes_i
    d = jnp.where(mask, flat - p, 0)
    val = jnp.where(mask, sim, -1.0)

    for b in range(17):
        s = 1 << b
        av = _flat_roll(val, s, -1.0)
        ad = _flat_roll(d, s, 0)
        bit = jnp.int32(s)
        arrives = (ad & bit) != 0
        leaves = (d & bit) != 0
        val = jnp.where(arrives, av, jnp.where(leaves, -1.0, val))
        d = jnp.where(arrives, ad - bit, jnp.where(leaves, 0, d))

    cand = val[:8, :]

    ci = lax.broadcasted_iota(jnp.int32, (8, D), 0) * D + \
        lax.broadcasted_iota(jnp.int32, (8, D), 1)
    lane_perm = {}
    for j in (1, 2, 4, 8, 16, 32, 64):
        a = lax.broadcasted_iota(jnp.int32, (D, D), 0)
        bb = lax.broadcasted_iota(jnp.int32, (D, D), 1)
        lane_perm[j] = ((a ^ j) == bb).astype(jnp.float32)
    row_perm = {}
    for mrows in (1, 2, 4):
        a = lax.broadcasted_iota(jnp.int32, (8, 8), 0)
        bb = lax.broadcasted_iota(jnp.int32, (8, 8), 1)
        row_perm[mrows] = ((a ^ mrows) == bb).astype(jnp.float32)

    k = 2
    while k <= TOP_K:
        j = k // 2
        while j >= 1:
            if j < D:
                prt = lax.dot_general(cand, lane_perm[j],
                                      (((1,), (0,)), ((), ())),
                                      preferred_element_type=jnp.float32)
            else:
                prt = lax.dot_general(row_perm[j // D], cand,
                                      (((1,), (0,)), ((), ())),
                                      preferred_element_type=jnp.float32)
            up = (ci & j) != 0
            desc = (ci & k) == 0
            take_max = up != desc
            cand = jnp.where(take_max, jnp.maximum(cand, prt),
                             jnp.minimum(cand, prt))
            j //= 2
        k *= 2

    topk = jnp.where(ci < m, cand, t_f)

    prod = w1_ref[...] * topk[None, :, :]
    h = jnp.sum(jnp.sum(prod, axis=2), axis=1).reshape(1, NH)
    h = (h + b1_ref[...]) * jnp.float32(1.0 / math.sqrt(1.0 + 1e-5))
    h = h * gam_ref[...] + bet_ref[...]
    h = jnp.maximum(h, 0.0)
    o = jnp.sum(h * w2_ref[...], axis=1, keepdims=True) + b2_ref[...]
    out_ref[...] = 1.0 / (1.0 + jnp.exp(-o))


NH = 16


def kernel(graph1, graph2, W1, b1, gamma, beta, W2, b2):
    sim = _distances(graph1, graph2).reshape(N)
    sim2d = jnp.concatenate([sim, jnp.zeros((PAD,), jnp.float32)]).reshape(
        ROWS, D)
    w1x = W1.T.reshape(NH, 8, D)
    return pl.pallas_call(
        _topk_mlp_body,
        out_shape=jax.ShapeDtypeStruct((1, 1), jnp.float32),
    )(sim2d, w1x, b1.reshape(1, NH), gamma.reshape(1, NH),
      beta.reshape(1, NH), W2.reshape(1, NH), b2.reshape(1, 1))


# speedup vs baseline: 3.2822x; 1.0717x over previous
import functools
import math

import jax
import jax.numpy as jnp
from jax import lax
from jax.experimental import pallas as pl

N = 100000
D = 128
TOP_K = 1024
ROWS = 800
PAD = ROWS * D - N
DIST_BLK = 4000
DIST_GRID = N // DIST_BLK


def _dist_body(g1_ref, g2_ref, out_ref):
    x = g1_ref[...] - g2_ref[...] + 1e-6
    ones = jnp.ones((1, D), jnp.float32)
    s = lax.dot_general(ones, x * x, (((1,), (1,)), ((), ())),
                        preferred_element_type=jnp.float32)
    out_ref[...] = jnp.sqrt(s)[None]


def _distances(g1, g2):
    return pl.pallas_call(
        _dist_body,
        grid=(DIST_GRID,),
        in_specs=[
            pl.BlockSpec((DIST_BLK, D), lambda i: (i, 0)),
            pl.BlockSpec((DIST_BLK, D), lambda i: (i, 0)),
        ],
        out_specs=pl.BlockSpec((1, 1, DIST_BLK), lambda i: (i, 0, 0)),
        out_shape=jax.ShapeDtypeStruct((DIST_GRID, 1, DIST_BLK), jnp.float32),
    )(g1, g2)


def _flat_roll(x, s, sent):
    r, c = x.shape
    if s >= c:
        m = s // c
        assert s % c == 0
        return jnp.concatenate(
            [x[m:, :], jnp.full((m, c), sent, x.dtype)], axis=0)
    nxt = jnp.concatenate([x[1:, :], jnp.full((1, c), sent, x.dtype)], axis=0)
    return jnp.concatenate([x[:, s:], nxt[:, :s]], axis=1)


def _topk_mlp_body(sim_ref, w1_ref, b1_ref, gam_ref, bet_ref, w2_ref, b2_ref,
                   out_ref):
    sim = sim_ref[...]
    bits = lax.bitcast_convert_type(sim, jnp.int32)

    def bs(_, carry):
        lo, hi = carry
        span = hi - lo + jnp.int32(1)
        mids = [lo + (span // 8) * jnp.int32(i) + ((span % 8) * jnp.int32(i)) // 8
                for i in range(1, 8)]
        cnts = [jnp.sum((bits >= mm).astype(jnp.int32)) for mm in mids]
        new_lo, new_hi = lo, mids[0] - 1
        for mm, cc in zip(mids, cnts):
            ge = cc >= TOP_K
            new_lo = jnp.where(ge, mm, new_lo)
        for i in range(6):
            take = jnp.logical_and(cnts[i] >= TOP_K, cnts[i + 1] < TOP_K)
            new_hi = jnp.where(take, mids[i + 1] - 1, new_hi)
        new_hi = jnp.where(cnts[6] >= TOP_K, hi, new_hi)
        return (new_lo, new_hi)

    lo, _ = lax.fori_loop(0, 13, bs, (jnp.int32(0), jnp.int32(0x7F800000)))
    t_f = lax.bitcast_convert_type(lo, jnp.float32)

    mask = bits > lo
    maskf = mask.astype(jnp.float32)
    m = jnp.sum(mask.astype(jnp.int32))

    li = lax.broadcasted_iota(jnp.int32, (D, D), 0)
    lj = lax.broadcasted_iota(jnp.int32, (D, D), 1)
    upper = (li <= lj).astype(jnp.float32)
    incl = lax.dot_general(maskf, upper, (((1,), (0,)), ((), ())),
                           preferred_element_type=jnp.float32)
    rowtot = incl[:, D - 1:D]
    ri = lax.broadcasted_iota(jnp.int32, (ROWS, ROWS), 0)
    rj = lax.broadcasted_iota(jnp.int32, (ROWS, ROWS), 1)
    ltri = (rj < ri).astype(jnp.float32)
    offs = lax.dot_general(ltri, rowtot, (((1,), (0,)), ((), ())),
                           preferred_element_type=jnp.float32)
    p = (offs + incl - maskf).astype(jnp.int32)

    rows_i = lax.broadcasted_iota(jnp.int32, (ROWS, D), 0)
    lanes_i = lax.broadcasted_iota(jnp.int32, (ROWS, D), 1)
    flat = rows_i * D + lanes_i
    d = jnp.where(mask, flat - p, 0)
    val = jnp.where(mask, sim, -1.0)

    for b in range(17):
        s = 1 << b
        av = _flat_roll(val, s, -1.0)
        ad = _flat_roll(d, s, 0)
        bit = jnp.int32(s)
        arrives = (ad & bit) != 0
        leaves = (d & bit) != 0
        val = jnp.where(arrives, av, jnp.where(leaves, -1.0, val))
        d = jnp.where(arrives, ad - bit, jnp.where(leaves, 0, d))

    cand = val[:8, :]

    ci = lax.broadcasted_iota(jnp.int32, (8, D), 0) * D + \
        lax.broadcasted_iota(jnp.int32, (8, D), 1)

    k = 2
    while k <= TOP_K:
        j = k // 2
        while j >= 1:
            up = (ci & j) != 0
            if j < D:
                prt = jnp.where(up, jnp.roll(cand, j, axis=1),
                                jnp.roll(cand, -j, axis=1))
            else:
                mr = j // D
                prt = jnp.where(up, jnp.roll(cand, mr, axis=0),
                                jnp.roll(cand, -mr, axis=0))
            desc = (ci & k) == 0
            take_max = up != desc
            cand = jnp.where(take_max, jnp.maximum(cand, prt),
                             jnp.minimum(cand, prt))
            j //= 2
        k *= 2

    topk = jnp.where(ci < m, cand, t_f)

    prod = w1_ref[...] * topk[None, :, :]
    h = jnp.sum(jnp.sum(prod, axis=2), axis=1).reshape(1, NH)
    h = (h + b1_ref[...]) * jnp.float32(1.0 / math.sqrt(1.0 + 1e-5))
    h = h * gam_ref[...] + bet_ref[...]
    h = jnp.maximum(h, 0.0)
    o = jnp.sum(h * w2_ref[...], axis=1, keepdims=True) + b2_ref[...]
    out_ref[...] = 1.0 / (1.0 + jnp.exp(-o))


NH = 16


def kernel(graph1, graph2, W1, b1, gamma, beta, W2, b2):
    sim = _distances(graph1, graph2).reshape(N)
    sim2d = jnp.concatenate([sim, jnp.zeros((PAD,), jnp.float32)]).reshape(
        ROWS, D)
    w1x = W1.T.reshape(NH, 8, D)
    return pl.pallas_call(
        _topk_mlp_body,
        out_shape=jax.ShapeDtypeStruct((1, 1), jnp.float32),
    )(sim2d, w1x, b1.reshape(1, NH), gamma.reshape(1, NH),
      beta.reshape(1, NH), W2.reshape(1, NH), b2.reshape(1, 1))


# speedup vs baseline: 3.4860x; 1.0621x over previous
import functools
import math

import jax
import jax.numpy as jnp
from jax import lax
from jax.experimental import pallas as pl

N = 100000
D = 128
TOP_K = 1024
ROWS = 800
PAD = ROWS * D - N
DIST_BLK = 10000
DIST_GRID = N // DIST_BLK


def _dist_body(g1_ref, g2_ref, out_ref):
    x = g1_ref[...] - g2_ref[...] + 1e-6
    ones = jnp.ones((1, D), jnp.float32)
    s = lax.dot_general(ones, x * x, (((1,), (1,)), ((), ())),
                        preferred_element_type=jnp.float32)
    out_ref[...] = jnp.sqrt(s)[None]


def _distances(g1, g2):
    return pl.pallas_call(
        _dist_body,
        grid=(DIST_GRID,),
        in_specs=[
            pl.BlockSpec((DIST_BLK, D), lambda i: (i, 0)),
            pl.BlockSpec((DIST_BLK, D), lambda i: (i, 0)),
        ],
        out_specs=pl.BlockSpec((1, 1, DIST_BLK), lambda i: (i, 0, 0)),
        out_shape=jax.ShapeDtypeStruct((DIST_GRID, 1, DIST_BLK), jnp.float32),
    )(g1, g2)


def _flat_roll(x, s, sent):
    r, c = x.shape
    if s >= c:
        m = s // c
        assert s % c == 0
        return jnp.concatenate(
            [x[m:, :], jnp.full((m, c), sent, x.dtype)], axis=0)
    nxt = jnp.concatenate([x[1:, :], jnp.full((1, c), sent, x.dtype)], axis=0)
    return jnp.concatenate([x[:, s:], nxt[:, :s]], axis=1)


def _topk_mlp_body(sim_ref, w1_ref, b1_ref, gam_ref, bet_ref, w2_ref, b2_ref,
                   out_ref):
    sim = sim_ref[...]
    bits = lax.bitcast_convert_type(sim, jnp.int32)

    def bs(_, carry):
        lo, hi = carry
        span = hi - lo + jnp.int32(1)
        mids = [lo + (span // 8) * jnp.int32(i) + ((span % 8) * jnp.int32(i)) // 8
                for i in range(1, 8)]
        cnts = [jnp.sum((bits >= mm).astype(jnp.int32)) for mm in mids]
        new_lo, new_hi = lo, mids[0] - 1
        for mm, cc in zip(mids, cnts):
            ge = cc >= TOP_K
            new_lo = jnp.where(ge, mm, new_lo)
        for i in range(6):
            take = jnp.logical_and(cnts[i] >= TOP_K, cnts[i + 1] < TOP_K)
            new_hi = jnp.where(take, mids[i + 1] - 1, new_hi)
        new_hi = jnp.where(cnts[6] >= TOP_K, hi, new_hi)
        return (new_lo, new_hi)

    lo, _ = lax.fori_loop(0, 13, bs, (jnp.int32(0), jnp.int32(0x7F800000)))
    t_f = lax.bitcast_convert_type(lo, jnp.float32)

    mask = bits > lo
    maskf = mask.astype(jnp.float32)
    m = jnp.sum(mask.astype(jnp.int32))

    li = lax.broadcasted_iota(jnp.int32, (D, D), 0)
    lj = lax.broadcasted_iota(jnp.int32, (D, D), 1)
    upper = (li <= lj).astype(jnp.float32)
    incl = lax.dot_general(maskf, upper, (((1,), (0,)), ((), ())),
                           preferred_element_type=jnp.float32)
    rowtot = incl[:, D - 1:D]
    ri = lax.broadcasted_iota(jnp.int32, (ROWS, ROWS), 0)
    rj = lax.broadcasted_iota(jnp.int32, (ROWS, ROWS), 1)
    ltri = (rj < ri).astype(jnp.float32)
    offs = lax.dot_general(ltri, rowtot, (((1,), (0,)), ((), ())),
                           preferred_element_type=jnp.float32)
    p = (offs + incl - maskf).astype(jnp.int32)

    rows_i = lax.broadcasted_iota(jnp.int32, (ROWS, D), 0)
    lanes_i = lax.broadcasted_iota(jnp.int32, (ROWS, D), 1)
    flat = rows_i * D + lanes_i
    d = jnp.where(mask, flat - p, 0)
    val = jnp.where(mask, sim, -1.0)

    for b in range(17):
        s = 1 << b
        av = _flat_roll(val, s, -1.0)
        ad = _flat_roll(d, s, 0)
        bit = jnp.int32(s)
        arrives = (ad & bit) != 0
        leaves = (d & bit) != 0
        val = jnp.where(arrives, av, jnp.where(leaves, -1.0, val))
        d = jnp.where(arrives, ad - bit, jnp.where(leaves, 0, d))

    cand = val[:8, :]

    ci = lax.broadcasted_iota(jnp.int32, (8, D), 0) * D + \
        lax.broadcasted_iota(jnp.int32, (8, D), 1)

    k = 2
    while k <= TOP_K:
        j = k // 2
        while j >= 1:
            up = (ci & j) != 0
            if j < D:
                prt = jnp.where(up, jnp.roll(cand, j, axis=1),
                                jnp.roll(cand, -j, axis=1))
            else:
                mr = j // D
                prt = jnp.where(up, jnp.roll(cand, mr, axis=0),
                                jnp.roll(cand, -mr, axis=0))
            desc = (ci & k) == 0
            take_max = up != desc
            cand = jnp.where(take_max, jnp.maximum(cand, prt),
                             jnp.minimum(cand, prt))
            j //= 2
        k *= 2

    topk = jnp.where(ci < m, cand, t_f)

    prod = w1_ref[...] * topk[None, :, :]
    h = jnp.sum(jnp.sum(prod, axis=2), axis=1).reshape(1, NH)
    h = (h + b1_ref[...]) * jnp.float32(1.0 / math.sqrt(1.0 + 1e-5))
    h = h * gam_ref[...] + bet_ref[...]
    h = jnp.maximum(h, 0.0)
    o = jnp.sum(h * w2_ref[...], axis=1, keepdims=True) + b2_ref[...]
    out_ref[...] = 1.0 / (1.0 + jnp.exp(-o))


NH = 16


def kernel(graph1, graph2, W1, b1, gamma, beta, W2, b2):
    sim = _distances(graph1, graph2).reshape(N)
    sim2d = jnp.concatenate([sim, jnp.zeros((PAD,), jnp.float32)]).reshape(
        ROWS, D)
    w1x = W1.T.reshape(NH, 8, D)
    return pl.pallas_call(
        _topk_mlp_body,
        out_shape=jax.ShapeDtypeStruct((1, 1), jnp.float32),
    )(sim2d, w1x, b1.reshape(1, NH), gamma.reshape(1, NH),
      beta.reshape(1, NH), W2.reshape(1, NH), b2.reshape(1, 1))
